# Initial kernel scaffold; baseline (speedup 1.0000x reference)
import functools

import jax
import jax.numpy as jnp
from jax import lax
from jax.experimental import pallas as pl
from jax.experimental.pallas import tpu as pltpu
from jax.experimental.pallas import tpu_sc as plsc

N_NODES = 10000
N_EDGES = 320000
D = 128

NC, NS, L = 2, 16, 16
NW = NC * NS
EPW = N_EDGES // NW
CH = 40
NCHUNK = EPW // CH
ROWS_PER_TILE = N_NODES // NS
ZROWS = 125
DEG_W = 16


def _sc_scatter(x, row3, col3):
    mesh = plsc.VectorSubcoreMesh(core_axis_name="c", subcore_axis_name="s")

    @functools.partial(
        pl.kernel,
        out_type=(
            jax.ShapeDtypeStruct((NC, N_NODES, D), jnp.float32),
            jax.ShapeDtypeStruct((NC, N_NODES, DEG_W), jnp.float32),
        ),
        mesh=mesh,
        scratch_types=[
            pltpu.VMEM((NCHUNK, CH), jnp.int32),
            pltpu.VMEM((NCHUNK, CH), jnp.int32),
            pltpu.VMEM((CH, D), jnp.float32),
            pltpu.VMEM((CH, D), jnp.float32),
            pltpu.VMEM((CH, DEG_W), jnp.float32),
            pltpu.VMEM((ZROWS, D), jnp.float32),
            pltpu.VMEM((ROWS_PER_TILE, DEG_W), jnp.float32),
            pltpu.VMEM_SHARED((N_NODES, D), jnp.float32),
            pltpu.VMEM_SHARED((N_NODES, DEG_W), jnp.float32),
            pltpu.SemaphoreType.DMA,
            pltpu.SemaphoreType.DMA,
        ],
    )
    def sc_kernel(x_hbm, row_hbm, col_hbm, s_out, deg_out,
                  row_v, col_v, buf0, buf1, ones_v, zbuf, dzbuf,
                  acc, dacc, sem0, sem1):
        c = lax.axis_index("c")
        s = lax.axis_index("s")
        wid = s * NC + c

        pltpu.sync_copy(row_hbm.at[wid], row_v)
        pltpu.sync_copy(col_hbm.at[wid], col_v)

        zero16 = jnp.zeros((L,), jnp.float32)
        one16 = jnp.ones((L,), jnp.float32)

        def fill_ones(i, carry):
            ones_v[i, :] = one16
            return carry
        lax.fori_loop(0, CH, fill_ones, 0)

        def fill_z(i, carry):
            for j in range(D // L):
                zbuf[i, pl.ds(j * L, L)] = zero16
            return carry
        lax.fori_loop(0, ZROWS, fill_z, 0)

        def fill_dz(i, carry):
            dzbuf[i, :] = zero16
            return carry
        lax.fori_loop(0, ROWS_PER_TILE, fill_dz, 0)

        base = s * ROWS_PER_TILE
        for kk in range(ROWS_PER_TILE // ZROWS):
            pltpu.sync_copy(zbuf, acc.at[pl.ds(base + kk * ZROWS, ZROWS)])
        pltpu.sync_copy(dzbuf, dacc.at[pl.ds(base, ROWS_PER_TILE)])

        plsc.subcore_barrier()

        pltpu.async_copy(x_hbm.at[col_v.at[0]], buf0, sem0)

        def chunk_pair(i, carry):
            cc = 2 * i
            pltpu.async_copy(x_hbm.at[col_v.at[cc + 1]], buf1, sem1)
            pltpu.make_async_copy(x_hbm.at[pl.ds(0, CH)], buf0, sem0).wait()
            pltpu.sync_copy(buf0, acc.at[row_v.at[cc]], add=True)
            pltpu.sync_copy(ones_v, dacc.at[row_v.at[cc]], add=True)

            @pl.when(cc + 2 < NCHUNK)
            def _():
                pltpu.async_copy(x_hbm.at[col_v.at[cc + 2]], buf0, sem0)

            pltpu.make_async_copy(x_hbm.at[pl.ds(0, CH)], buf1, sem1).wait()
            pltpu.sync_copy(buf1, acc.at[row_v.at[cc + 1]], add=True)
            pltpu.sync_copy(ones_v, dacc.at[row_v.at[cc + 1]], add=True)
            return carry
        lax.fori_loop(0, NCHUNK // 2, chunk_pair, 0)

        plsc.subcore_barrier()

        pltpu.sync_copy(acc.at[pl.ds(base, ROWS_PER_TILE)],
                        s_out.at[c, pl.ds(base, ROWS_PER_TILE)])
        pltpu.sync_copy(dacc.at[pl.ds(base, ROWS_PER_TILE)],
                        deg_out.at[c, pl.ds(base, ROWS_PER_TILE)])

    return sc_kernel(x, row3, col3)


def _tc_combine(x, W, b2, s2, d2):
    BLK = 1000

    def body(x_ref, w_ref, b_ref, s_ref, d_ref, o_ref):
        S = s_ref[0] + s_ref[1]
        deg = d_ref[0, :, 0:1] + d_ref[1, :, 0:1]
        Wa = w_ref[0:D, :]
        Wb = w_ref[D:2 * D, :]
        xs = x_ref[...] * deg
        o_ref[...] = (
            jnp.dot(xs, Wa - Wb, preferred_element_type=jnp.float32)
            + jnp.dot(S, Wb, preferred_element_type=jnp.float32)
            + deg * b_ref[...]
        )

    return pl.pallas_call(
        body,
        grid=(N_NODES // BLK,),
        in_specs=[
            pl.BlockSpec((BLK, D), lambda i: (i, 0)),
            pl.BlockSpec((2 * D, D), lambda i: (0, 0)),
            pl.BlockSpec((1, D), lambda i: (0, 0)),
            pl.BlockSpec((NC, BLK, D), lambda i: (0, i, 0)),
            pl.BlockSpec((NC, BLK, DEG_W), lambda i: (0, i, 0)),
        ],
        out_specs=pl.BlockSpec((BLK, D), lambda i: (i, 0)),
        out_shape=jax.ShapeDtypeStruct((N_NODES, D), jnp.float32),
    )(x, W, b2, s2, d2)


@jax.jit
def kernel(x, edge_index, W, b):
    ei = edge_index.astype(jnp.int32)
    row3 = ei[0].reshape(NW, NCHUNK, CH)
    col3 = ei[1].reshape(NW, NCHUNK, CH)
    s2, d2 = _sc_scatter(x, row3, col3)
    return _tc_combine(x, W, b.reshape(1, D), s2, d2)


# speedup vs baseline: 5.0730x; 5.0730x over previous
import functools

import jax
import jax.numpy as jnp
from jax import lax
from jax.experimental import pallas as pl
from jax.experimental.pallas import tpu as pltpu
from jax.experimental.pallas import tpu_sc as plsc

N_NODES = 10000
N_EDGES = 320000
D = 128

NC, NS, L = 2, 16, 16
NW = NC * NS
EPW = N_EDGES // NW
CH = 40
NCHUNK = EPW // CH
N_PAD = 10240
ROWS_PER_TILE = N_PAD // NS


def _sc_scatter(x, row1d, col1d):
    mesh = plsc.VectorSubcoreMesh(core_axis_name="c", subcore_axis_name="s")

    @functools.partial(
        pl.kernel,
        out_type=(
            jax.ShapeDtypeStruct((NC, N_PAD, D), jnp.float32),
            jax.ShapeDtypeStruct((NC, N_PAD), jnp.float32),
        ),
        mesh=mesh,
        scratch_types=[
            pltpu.VMEM((CH,), jnp.int32),
            pltpu.VMEM((CH,), jnp.int32),
            pltpu.VMEM((CH, D), jnp.float32),
            pltpu.VMEM((48,), jnp.float32),
            pltpu.VMEM((48,), jnp.float32),
            pltpu.VMEM_SHARED((N_PAD, D), jnp.float32),
            pltpu.VMEM_SHARED((N_PAD,), jnp.float32),
            pltpu.SemaphoreType.DMA,
        ],
    )
    def sc_kernel(x_hbm, row_hbm, col_hbm, s_out, deg_out,
                  rowb, colb, buf0, ones_v, z40, acc, dacc, sem0):
        c = lax.axis_index("c")
        s = lax.axis_index("s")
        wid = s * NC + c
        ebase = wid * EPW

        zero16 = jnp.zeros((L,), jnp.float32)
        one16 = jnp.ones((L,), jnp.float32)

        for i in range(3):
            ones_v[pl.ds(i * L, L)] = one16
            z40[pl.ds(i * L, L)] = zero16

        def fill_buf(i, carry):
            for j in range(D // L):
                buf0[i, pl.ds(j * L, L)] = zero16
            return carry
        lax.fori_loop(0, CH, fill_buf, 0)

        base = s * ROWS_PER_TILE
        for kk in range(ROWS_PER_TILE // CH):
            pltpu.sync_copy(buf0, acc.at[pl.ds(base + kk * CH, CH)])
            pltpu.sync_copy(z40.at[pl.ds(0, CH)], dacc.at[pl.ds(base + kk * CH, CH)])

        plsc.subcore_barrier()

        def chunk_body(cc, carry):
            eb = ebase + cc * CH
            pltpu.sync_copy(row_hbm.at[pl.ds(eb, CH)], rowb)
            pltpu.sync_copy(col_hbm.at[pl.ds(eb, CH)], colb)
            pltpu.async_copy(x_hbm.at[colb], buf0, sem0).wait()
            pltpu.sync_copy(buf0, acc.at[rowb], add=True)
            pltpu.sync_copy(ones_v.at[pl.ds(0, CH)], dacc.at[rowb], add=True)
            return carry
        lax.fori_loop(0, NCHUNK, chunk_body, 0)

        plsc.subcore_barrier()

        pltpu.sync_copy(acc.at[pl.ds(base, ROWS_PER_TILE)],
                        s_out.at[c, pl.ds(base, ROWS_PER_TILE)])
        pltpu.sync_copy(dacc.at[pl.ds(base, ROWS_PER_TILE)],
                        deg_out.at[c, pl.ds(base, ROWS_PER_TILE)])

    return sc_kernel(x, row1d, col1d)


def _tc_combine(x, W, b2, s2, deg_col):
    BLK = 1000

    def body(x_ref, w_ref, b_ref, s_ref, d_ref, o_ref):
        S = s_ref[0] + s_ref[1]
        deg = d_ref[...]
        Wa = w_ref[0:D, :]
        Wb = w_ref[D:2 * D, :]
        xs = x_ref[...] * deg
        o_ref[...] = (
            jnp.dot(xs, Wa - Wb, preferred_element_type=jnp.float32)
            + jnp.dot(S, Wb, preferred_element_type=jnp.float32)
            + deg * b_ref[...]
        )

    return pl.pallas_call(
        body,
        grid=(N_NODES // BLK,),
        in_specs=[
            pl.BlockSpec((BLK, D), lambda i: (i, 0)),
            pl.BlockSpec((2 * D, D), lambda i: (0, 0)),
            pl.BlockSpec((1, D), lambda i: (0, 0)),
            pl.BlockSpec((NC, BLK, D), lambda i: (0, i, 0)),
            pl.BlockSpec((BLK, 1), lambda i: (i, 0)),
        ],
        out_specs=pl.BlockSpec((BLK, D), lambda i: (i, 0)),
        out_shape=jax.ShapeDtypeStruct((N_NODES, D), jnp.float32),
    )(x, W, b2, s2, deg_col)


@jax.jit
def kernel(x, edge_index, W, b):
    ei = edge_index.astype(jnp.int32)
    s2, d2 = _sc_scatter(x, ei[0], ei[1])
    deg_col = (d2[0, :N_NODES] + d2[1, :N_NODES]).reshape(N_NODES, 1)
    return _tc_combine(x, W, b.reshape(1, D), s2, deg_col)


# speedup vs baseline: 15.1609x; 2.9886x over previous
import functools

import jax
import jax.numpy as jnp
from jax import lax
from jax.experimental import pallas as pl
from jax.experimental.pallas import tpu as pltpu
from jax.experimental.pallas import tpu_sc as plsc

N_NODES = 10000
N_EDGES = 320000
D = 128

NC, NS, L = 2, 16, 16
NW = NC * NS
EPW = N_EDGES // NW
CH = 80
NCHUNK = EPW // CH
NB = 4
NG = NCHUNK // NB - 1
N_PAD = 10240
ROWS_PER_TILE = N_PAD // NS


def _sc_scatter(x, row1d, col1d):
    mesh = plsc.VectorSubcoreMesh(core_axis_name="c", subcore_axis_name="s")

    @functools.partial(
        pl.kernel,
        out_type=(
            jax.ShapeDtypeStruct((NC, N_PAD, D), jnp.float32),
            jax.ShapeDtypeStruct((NC, N_PAD), jnp.float32),
        ),
        mesh=mesh,
        scratch_types=(
            [pltpu.VMEM((CH,), jnp.int32) for _ in range(NB)]
            + [pltpu.VMEM((CH,), jnp.int32) for _ in range(NB)]
            + [pltpu.VMEM((CH, D), jnp.float32) for _ in range(NB)]
            + [
                pltpu.VMEM((CH,), jnp.float32),
                pltpu.VMEM((CH,), jnp.float32),
                pltpu.VMEM_SHARED((N_PAD, D), jnp.float32),
                pltpu.VMEM_SHARED((N_PAD,), jnp.float32),
            ]
            + [pltpu.SemaphoreType.DMA for _ in range(3 * NB)]
        ),
    )
    def sc_kernel(x_hbm, row_hbm, col_hbm, s_out, deg_out, *scr):
        rowb = scr[0:NB]
        colb = scr[NB:2 * NB]
        gbuf = scr[2 * NB:3 * NB]
        ones_v, z80, acc, dacc = scr[3 * NB:3 * NB + 4]
        semi = scr[3 * NB + 4:4 * NB + 4]
        semg = scr[4 * NB + 4:5 * NB + 4]
        sems = scr[5 * NB + 4:6 * NB + 4]

        c = lax.axis_index("c")
        s = lax.axis_index("s")
        wid = s * NC + c
        ebase = wid * EPW

        zero16 = jnp.zeros((L,), jnp.float32)
        one16 = jnp.ones((L,), jnp.float32)

        for i in range(CH // L):
            ones_v[pl.ds(i * L, L)] = one16
            z80[pl.ds(i * L, L)] = zero16

        def fill_buf(i, carry):
            for j in range(D // L):
                gbuf[0][i, pl.ds(j * L, L)] = zero16
            return carry
        lax.fori_loop(0, CH, fill_buf, 0)

        base = s * ROWS_PER_TILE
        for kk in range(ROWS_PER_TILE // CH):
            pltpu.sync_copy(gbuf[0], acc.at[pl.ds(base + kk * CH, CH)])
            pltpu.sync_copy(z80, dacc.at[pl.ds(base + kk * CH, CH)])

        plsc.subcore_barrier()

        def idx_start(cc, b):
            eb = ebase + cc * CH
            pltpu.async_copy(row_hbm.at[pl.ds(eb, CH)], rowb[b], semi[b])
            pltpu.async_copy(col_hbm.at[pl.ds(eb, CH)], colb[b], semi[b])

        def idx_wait(cc, b):
            eb = ebase + cc * CH
            pltpu.make_async_copy(row_hbm.at[pl.ds(eb, CH)], rowb[b], semi[b]).wait()
            pltpu.make_async_copy(col_hbm.at[pl.ds(eb, CH)], colb[b], semi[b]).wait()

        def gather_start(b):
            pltpu.async_copy(x_hbm.at[colb[b]], gbuf[b], semg[b])

        def gather_wait(b):
            pltpu.make_async_copy(x_hbm.at[colb[b]], gbuf[b], semg[b]).wait()

        def scatter_start(b):
            pltpu.async_copy(gbuf[b], acc.at[rowb[b]], sems[b], add=True)
            pltpu.async_copy(ones_v, dacc.at[rowb[b]], sems[b], add=True)

        def scatter_wait(b):
            pltpu.make_async_copy(gbuf[b], acc.at[rowb[b]], sems[b]).wait()
            pltpu.make_async_copy(ones_v, dacc.at[rowb[b]], sems[b]).wait()

        for b in range(NB):
            idx_start(b, b)

        def group_body(g, carry):
            cg = g * NB
            for b in range(NB):
                idx_wait(cg + b, b)
                gather_start(b)
            for b in range(NB):
                gather_wait(b)
                scatter_start(b)
            for b in range(NB):
                scatter_wait(b)
                idx_start(cg + NB + b, b)
            return carry
        lax.fori_loop(0, NG, group_body, 0)

        ce = NG * NB
        for b in range(NB):
            idx_wait(ce + b, b)
            gather_start(b)
        for b in range(NB):
            gather_wait(b)
            scatter_start(b)
        scatter_wait(0)
        pltpu.sync_copy(row_hbm.at[pl.ds(ebase + (NCHUNK - 1) * CH, CH)], rowb[0])
        pltpu.sync_copy(col_hbm.at[pl.ds(ebase + (NCHUNK - 1) * CH, CH)], colb[0])
        pltpu.async_copy(x_hbm.at[colb[0]], gbuf[0], semg[0]).wait()
        pltpu.sync_copy(gbuf[0], acc.at[rowb[0]], add=True)
        pltpu.sync_copy(ones_v, dacc.at[rowb[0]], add=True)
        for b in range(1, NB):
            scatter_wait(b)

        plsc.subcore_barrier()

        pltpu.sync_copy(acc.at[pl.ds(base, ROWS_PER_TILE)],
                        s_out.at[c, pl.ds(base, ROWS_PER_TILE)])
        pltpu.sync_copy(dacc.at[pl.ds(base, ROWS_PER_TILE)],
                        deg_out.at[c, pl.ds(base, ROWS_PER_TILE)])

    return sc_kernel(x, row1d, col1d)


def _tc_combine(x, W, b2, s2, deg_col):
    BLK = 1000

    def body(x_ref, w_ref, b_ref, s_ref, d_ref, o_ref):
        S = s_ref[0] + s_ref[1]
        deg = d_ref[...]
        Wa = w_ref[0:D, :]
        Wb = w_ref[D:2 * D, :]
        xs = x_ref[...] * deg
        o_ref[...] = (
            jnp.dot(xs, Wa - Wb, preferred_element_type=jnp.float32)
            + jnp.dot(S, Wb, preferred_element_type=jnp.float32)
            + deg * b_ref[...]
        )

    return pl.pallas_call(
        body,
        grid=(N_NODES // BLK,),
        in_specs=[
            pl.BlockSpec((BLK, D), lambda i: (i, 0)),
            pl.BlockSpec((2 * D, D), lambda i: (0, 0)),
            pl.BlockSpec((1, D), lambda i: (0, 0)),
            pl.BlockSpec((NC, BLK, D), lambda i: (0, i, 0)),
            pl.BlockSpec((BLK, 1), lambda i: (i, 0)),
        ],
        out_specs=pl.BlockSpec((BLK, D), lambda i: (i, 0)),
        out_shape=jax.ShapeDtypeStruct((N_NODES, D), jnp.float32),
    )(x, W, b2, s2, deg_col)


@jax.jit
def kernel(x, edge_index, W, b):
    ei = edge_index.astype(jnp.int32)
    s2, d2 = _sc_scatter(x, ei[0], ei[1])
    deg_col = (d2[0, :N_NODES] + d2[1, :N_NODES]).reshape(N_NODES, 1)
    return _tc_combine(x, W, b.reshape(1, D), s2, deg_col)


# speedup vs baseline: 15.2997x; 1.0092x over previous
import functools

import jax
import jax.numpy as jnp
from jax import lax
from jax.experimental import pallas as pl
from jax.experimental.pallas import tpu as pltpu
from jax.experimental.pallas import tpu_sc as plsc

N_NODES = 10000
N_EDGES = 320000
D = 128

NC, NS, L = 2, 16, 16
NW = NC * NS
EPW = N_EDGES // NW
CH = 80
NCHUNK = EPW // CH
NB = 4
NG = NCHUNK // NB - 1
CSLAB = 64
NG_A = CSLAB // NB
N_PAD = 10240
ROWS_PER_TILE = N_PAD // NS


def _sc_scatter(x, row1d, col1d):
    mesh = plsc.VectorSubcoreMesh(core_axis_name="c", subcore_axis_name="s")

    @functools.partial(
        pl.kernel,
        out_type=(
            jax.ShapeDtypeStruct((NC, N_PAD, D), jnp.float32),
            jax.ShapeDtypeStruct((NC, N_PAD), jnp.float32),
        ),
        mesh=mesh,
        scratch_types=(
            [pltpu.VMEM((CH,), jnp.int32) for _ in range(NB)]
            + [pltpu.VMEM((CSLAB * CH,), jnp.int32)]
            + [pltpu.VMEM((CH, D), jnp.float32) for _ in range(NB)]
            + [
                pltpu.VMEM((CH,), jnp.float32),
                pltpu.VMEM((CH,), jnp.float32),
                pltpu.VMEM_SHARED((N_PAD, D), jnp.float32),
                pltpu.VMEM_SHARED((N_PAD,), jnp.float32),
            ]
            + [pltpu.SemaphoreType.DMA for _ in range(3 * NB)]
        ),
    )
    def sc_kernel(x_hbm, row_hbm, col_hbm, s_out, deg_out, *scr):
        rowb = scr[0:NB]
        colslab = scr[NB]
        gbuf = scr[NB + 1:2 * NB + 1]
        ones_v, z80, acc, dacc = scr[2 * NB + 1:2 * NB + 5]
        semi = scr[2 * NB + 5:3 * NB + 5]
        semg = scr[3 * NB + 5:4 * NB + 5]
        sems = scr[4 * NB + 5:5 * NB + 5]

        c = lax.axis_index("c")
        s = lax.axis_index("s")
        wid = s * NC + c
        ebase = wid * EPW

        zero16 = jnp.zeros((L,), jnp.float32)
        one16 = jnp.ones((L,), jnp.float32)

        for i in range(CH // L):
            ones_v[pl.ds(i * L, L)] = one16
            z80[pl.ds(i * L, L)] = zero16

        def fill_buf(i, carry):
            for j in range(D // L):
                gbuf[0][i, pl.ds(j * L, L)] = zero16
            return carry
        lax.fori_loop(0, CH, fill_buf, 0)

        pltpu.async_copy(col_hbm.at[pl.ds(ebase, CSLAB * CH)], colslab, semg[0])

        base = s * ROWS_PER_TILE
        for kk in range(ROWS_PER_TILE // CH):
            pltpu.async_copy(gbuf[0], acc.at[pl.ds(base + kk * CH, CH)],
                             semg[1 + (kk % 2)])
            pltpu.async_copy(z80, dacc.at[pl.ds(base + kk * CH, CH)],
                             semg[3])
        pltpu.make_async_copy(col_hbm.at[pl.ds(ebase, CSLAB * CH)], colslab,
                              semg[0]).wait()
        for kk in range(ROWS_PER_TILE // CH):
            pltpu.make_async_copy(gbuf[0], acc.at[pl.ds(base, CH)],
                                  semg[1 + (kk % 2)]).wait()
            pltpu.make_async_copy(z80, dacc.at[pl.ds(base, CH)],
                                  semg[3]).wait()

        plsc.subcore_barrier()

        def idx_start(cc, b):
            eb = ebase + cc * CH
            pltpu.async_copy(row_hbm.at[pl.ds(eb, CH)], rowb[b], semi[b])

        def idx_wait(cc, b):
            eb = ebase + cc * CH
            pltpu.make_async_copy(row_hbm.at[pl.ds(eb, CH)], rowb[b], semi[b]).wait()

        def cidx(cc, coff):
            return colslab.at[pl.ds((cc - coff) * CH, CH)]

        def gather_start(cc, b, coff):
            pltpu.async_copy(x_hbm.at[cidx(cc, coff)], gbuf[b], semg[b])

        def gather_wait(cc, b, coff):
            pltpu.make_async_copy(x_hbm.at[cidx(cc, coff)], gbuf[b], semg[b]).wait()

        def scatter_start(b):
            pltpu.async_copy(gbuf[b], acc.at[rowb[b]], sems[b], add=True)
            pltpu.async_copy(ones_v, dacc.at[rowb[b]], sems[b], add=True)

        def scatter_wait(b):
            pltpu.make_async_copy(gbuf[b], acc.at[rowb[b]], sems[b]).wait()
            pltpu.make_async_copy(ones_v, dacc.at[rowb[b]], sems[b]).wait()

        for b in range(NB):
            idx_start(b, b)

        def make_group_body(coff):
            def group_body(g, carry):
                cg = g * NB
                for b in range(NB):
                    idx_wait(cg + b, b)
                    gather_start(cg + b, b, coff)
                for b in range(NB):
                    gather_wait(cg + b, b, coff)
                    scatter_start(b)
                for b in range(NB):
                    scatter_wait(b)
                    idx_start(cg + NB + b, b)
                return carry
            return group_body

        lax.fori_loop(0, NG_A, make_group_body(0), 0)
        nb_rest = (NCHUNK - CSLAB) * CH
        pltpu.sync_copy(col_hbm.at[pl.ds(ebase + CSLAB * CH, nb_rest)],
                        colslab.at[pl.ds(0, nb_rest)])
        lax.fori_loop(NG_A, NG, make_group_body(CSLAB), 0)

        ce = NG * NB
        for b in range(NB):
            idx_wait(ce + b, b)
            gather_start(ce + b, b, CSLAB)
        for b in range(NB):
            gather_wait(ce + b, b, CSLAB)
            scatter_start(b)
        scatter_wait(0)
        pltpu.sync_copy(row_hbm.at[pl.ds(ebase + (NCHUNK - 1) * CH, CH)], rowb[0])
        pltpu.async_copy(x_hbm.at[cidx(NCHUNK - 1, CSLAB)], gbuf[0], semg[0]).wait()
        pltpu.sync_copy(gbuf[0], acc.at[rowb[0]], add=True)
        pltpu.sync_copy(ones_v, dacc.at[rowb[0]], add=True)
        for b in range(1, NB):
            scatter_wait(b)

        plsc.subcore_barrier()

        pltpu.sync_copy(acc.at[pl.ds(base, ROWS_PER_TILE)],
                        s_out.at[c, pl.ds(base, ROWS_PER_TILE)])
        pltpu.sync_copy(dacc.at[pl.ds(base, ROWS_PER_TILE)],
                        deg_out.at[c, pl.ds(base, ROWS_PER_TILE)])

    return sc_kernel(x, row1d, col1d)


def _tc_combine(x, W, b2, s2, deg_col):
    BLK = 1000

    def body(x_ref, w_ref, b_ref, s_ref, d_ref, o_ref):
        S = s_ref[0] + s_ref[1]
        deg = d_ref[...]
        Wa = w_ref[0:D, :]
        Wb = w_ref[D:2 * D, :]
        xs = x_ref[...] * deg
        o_ref[...] = (
            jnp.dot(xs, Wa - Wb, preferred_element_type=jnp.float32)
            + jnp.dot(S, Wb, preferred_element_type=jnp.float32)
            + deg * b_ref[...]
        )

    return pl.pallas_call(
        body,
        grid=(N_NODES // BLK,),
        in_specs=[
            pl.BlockSpec((BLK, D), lambda i: (i, 0)),
            pl.BlockSpec((2 * D, D), lambda i: (0, 0)),
            pl.BlockSpec((1, D), lambda i: (0, 0)),
            pl.BlockSpec((NC, BLK, D), lambda i: (0, i, 0)),
            pl.BlockSpec((BLK, 1), lambda i: (i, 0)),
        ],
        out_specs=pl.BlockSpec((BLK, D), lambda i: (i, 0)),
        out_shape=jax.ShapeDtypeStruct((N_NODES, D), jnp.float32),
    )(x, W, b2, s2, deg_col)


@jax.jit
def kernel(x, edge_index, W, b):
    ei = edge_index.astype(jnp.int32)
    s2, d2 = _sc_scatter(x, ei[0], ei[1])
    deg_col = (d2[0, :N_NODES] + d2[1, :N_NODES]).reshape(N_NODES, 1)
    return _tc_combine(x, W, b.reshape(1, D), s2, deg_col)


# speedup vs baseline: 16.1488x; 1.0555x over previous
import functools

import jax
import jax.numpy as jnp
from jax import lax
from jax.experimental import pallas as pl
from jax.experimental.pallas import tpu as pltpu
from jax.experimental.pallas import tpu_sc as plsc

N_NODES = 10000
N_EDGES = 320000
D = 128

NC, NS, L = 2, 16, 16
NW = NC * NS
EPW = N_EDGES // NW
CH = 80
NCHUNK = EPW // CH
NB = 4
NG = NCHUNK // NB - 1
CSLAB = 64
NG_A = CSLAB // NB
N_PAD = 10240
ROWS_PER_TILE = N_PAD // NS


def _sc_scatter(x, eflat):
    mesh = plsc.VectorSubcoreMesh(core_axis_name="c", subcore_axis_name="s")

    @functools.partial(
        pl.kernel,
        out_type=(
            jax.ShapeDtypeStruct((NC, N_PAD, D), jnp.float32),
            jax.ShapeDtypeStruct((NC, N_PAD), jnp.float32),
        ),
        mesh=mesh,
        scratch_types=(
            [pltpu.VMEM((CH,), jnp.int32) for _ in range(NB)]
            + [pltpu.VMEM((CSLAB * CH,), jnp.int32)]
            + [pltpu.VMEM((CH, D), jnp.float32) for _ in range(NB)]
            + [
                pltpu.VMEM((CH,), jnp.float32),
                pltpu.VMEM((CH,), jnp.float32),
                pltpu.VMEM_SHARED((N_PAD, D), jnp.float32),
                pltpu.VMEM_SHARED((N_PAD,), jnp.float32),
            ]
            + [pltpu.SemaphoreType.DMA for _ in range(3 * NB)]
        ),
    )
    def sc_kernel(x_hbm, e_hbm, s_out, deg_out, *scr):
        rowb = scr[0:NB]
        colslab = scr[NB]
        gbuf = scr[NB + 1:2 * NB + 1]
        ones_v, z80, acc, dacc = scr[2 * NB + 1:2 * NB + 5]
        semi = scr[2 * NB + 5:3 * NB + 5]
        semg = scr[3 * NB + 5:4 * NB + 5]
        sems = scr[4 * NB + 5:5 * NB + 5]

        c = lax.axis_index("c")
        s = lax.axis_index("s")
        wid = s * NC + c
        ebase = wid * EPW

        zero16 = jnp.zeros((L,), jnp.float32)
        one16 = jnp.ones((L,), jnp.float32)

        for i in range(CH // L):
            ones_v[pl.ds(i * L, L)] = one16
            z80[pl.ds(i * L, L)] = zero16

        def fill_buf(i, carry):
            for j in range(D // L):
                gbuf[0][i, pl.ds(j * L, L)] = zero16
            return carry
        lax.fori_loop(0, CH, fill_buf, 0)

        cbase = N_EDGES + ebase
        pltpu.async_copy(e_hbm.at[pl.ds(cbase, CSLAB * CH)], colslab, semg[0])

        base = s * ROWS_PER_TILE
        for kk in range(ROWS_PER_TILE // CH):
            pltpu.async_copy(gbuf[0], acc.at[pl.ds(base + kk * CH, CH)],
                             semg[1 + (kk % 2)])
            pltpu.async_copy(z80, dacc.at[pl.ds(base + kk * CH, CH)],
                             semg[3])
        pltpu.make_async_copy(e_hbm.at[pl.ds(cbase, CSLAB * CH)], colslab,
                              semg[0]).wait()
        for kk in range(ROWS_PER_TILE // CH):
            pltpu.make_async_copy(gbuf[0], acc.at[pl.ds(base, CH)],
                                  semg[1 + (kk % 2)]).wait()
            pltpu.make_async_copy(z80, dacc.at[pl.ds(base, CH)],
                                  semg[3]).wait()

        plsc.subcore_barrier()

        def idx_start(cc, b):
            eb = ebase + cc * CH
            pltpu.async_copy(e_hbm.at[pl.ds(eb, CH)], rowb[b], semi[b])

        def idx_wait(cc, b):
            eb = ebase + cc * CH
            pltpu.make_async_copy(e_hbm.at[pl.ds(eb, CH)], rowb[b], semi[b]).wait()

        def cidx(cc, coff):
            return colslab.at[pl.ds((cc - coff) * CH, CH)]

        def gather_start(cc, b, coff):
            pltpu.async_copy(x_hbm.at[cidx(cc, coff)], gbuf[b], semg[b])

        def gather_wait(cc, b, coff):
            pltpu.make_async_copy(x_hbm.at[cidx(cc, coff)], gbuf[b], semg[b]).wait()

        def scatter_start(b):
            pltpu.async_copy(gbuf[b], acc.at[rowb[b]], sems[b], add=True)
            pltpu.async_copy(ones_v, dacc.at[rowb[b]], sems[b], add=True)

        def scatter_wait(b):
            pltpu.make_async_copy(gbuf[b], acc.at[rowb[b]], sems[b]).wait()
            pltpu.make_async_copy(ones_v, dacc.at[rowb[b]], sems[b]).wait()

        for b in range(NB):
            idx_start(b, b)

        def make_group_body(coff):
            def group_body(g, carry):
                cg = g * NB
                for b in range(NB):
                    idx_wait(cg + b, b)
                    gather_start(cg + b, b, coff)
                for b in range(NB):
                    gather_wait(cg + b, b, coff)
                    scatter_start(b)
                for b in range(NB):
                    scatter_wait(b)
                    idx_start(cg + NB + b, b)
                return carry
            return group_body

        lax.fori_loop(0, NG_A, make_group_body(0), 0)
        nb_rest = (NCHUNK - CSLAB) * CH
        pltpu.sync_copy(e_hbm.at[pl.ds(cbase + CSLAB * CH, nb_rest)],
                        colslab.at[pl.ds(0, nb_rest)])
        lax.fori_loop(NG_A, NG, make_group_body(CSLAB), 0)

        ce = NG * NB
        for b in range(NB):
            idx_wait(ce + b, b)
            gather_start(ce + b, b, CSLAB)
        for b in range(NB):
            gather_wait(ce + b, b, CSLAB)
            scatter_start(b)
        scatter_wait(0)
        pltpu.sync_copy(e_hbm.at[pl.ds(ebase + (NCHUNK - 1) * CH, CH)], rowb[0])
        pltpu.async_copy(x_hbm.at[cidx(NCHUNK - 1, CSLAB)], gbuf[0], semg[0]).wait()
        pltpu.sync_copy(gbuf[0], acc.at[rowb[0]], add=True)
        pltpu.sync_copy(ones_v, dacc.at[rowb[0]], add=True)
        for b in range(1, NB):
            scatter_wait(b)

        plsc.subcore_barrier()

        pltpu.sync_copy(acc.at[pl.ds(base, ROWS_PER_TILE)],
                        s_out.at[c, pl.ds(base, ROWS_PER_TILE)])
        pltpu.sync_copy(dacc.at[pl.ds(base, ROWS_PER_TILE)],
                        deg_out.at[c, pl.ds(base, ROWS_PER_TILE)])

    return sc_kernel(x, eflat)


def _tc_precompute(x, W, b2):
    BLK = 1000

    def body(x_ref, w_ref, b_ref, p_ref):
        Wa = w_ref[0:D, :]
        Wb = w_ref[D:2 * D, :]
        p_ref[...] = jnp.dot(x_ref[...], Wa - Wb,
                             preferred_element_type=jnp.float32) + b_ref[...]

    return pl.pallas_call(
        body,
        grid=(N_NODES // BLK,),
        in_specs=[
            pl.BlockSpec((BLK, D), lambda i: (i, 0)),
            pl.BlockSpec((2 * D, D), lambda i: (0, 0)),
            pl.BlockSpec((1, D), lambda i: (0, 0)),
        ],
        out_specs=pl.BlockSpec((BLK, D), lambda i: (i, 0)),
        out_shape=jax.ShapeDtypeStruct((N_NODES, D), jnp.float32),
    )(x, W, b2)


def _tc_combine(P, W, s2, deg_col):
    BLK = 1000

    def body(p_ref, w_ref, s_ref, d_ref, o_ref):
        S = s_ref[0] + s_ref[1]
        deg = d_ref[...]
        Wb = w_ref[D:2 * D, :]
        o_ref[...] = deg * p_ref[...] + jnp.dot(
            S, Wb, preferred_element_type=jnp.float32)

    return pl.pallas_call(
        body,
        grid=(N_NODES // BLK,),
        in_specs=[
            pl.BlockSpec((BLK, D), lambda i: (i, 0)),
            pl.BlockSpec((2 * D, D), lambda i: (0, 0)),
            pl.BlockSpec((NC, BLK, D), lambda i: (0, i, 0)),
            pl.BlockSpec((BLK, 1), lambda i: (i, 0)),
        ],
        out_specs=pl.BlockSpec((BLK, D), lambda i: (i, 0)),
        out_shape=jax.ShapeDtypeStruct((N_NODES, D), jnp.float32),
    )(P, W, s2, deg_col)


@jax.jit
def kernel(x, edge_index, W, b):
    eflat = edge_index.astype(jnp.int32).reshape(2 * N_EDGES)
    s2, d2 = _sc_scatter(x, eflat)
    P = _tc_precompute(x, W, b.reshape(1, D))
    deg_col = (d2[0, :N_NODES] + d2[1, :N_NODES]).reshape(N_NODES, 1)
    return _tc_combine(P, W, s2, deg_col)


# speedup vs baseline: 16.4486x; 1.0186x over previous
import functools

import jax
import jax.numpy as jnp
from jax import lax
from jax.experimental import pallas as pl
from jax.experimental.pallas import tpu as pltpu
from jax.experimental.pallas import tpu_sc as plsc

N_NODES = 10000
N_EDGES = 320000
D = 128

NC, NS, L = 2, 16, 16
NW = NC * NS
EPW = N_EDGES // NW
CH = 40
NCHUNK = EPW // CH
NB = 8
NG = NCHUNK // NB - 1
CSLAB = 128
NG_A = CSLAB // NB
ONES_LEN = ((CH + L - 1) // L) * L if False else ((CH + 15) // 16) * 16
N_PAD = 10240
ROWS_PER_TILE = N_PAD // NS


def _sc_scatter(x, eflat):
    mesh = plsc.VectorSubcoreMesh(core_axis_name="c", subcore_axis_name="s")

    @functools.partial(
        pl.kernel,
        out_type=(
            jax.ShapeDtypeStruct((NC, N_PAD, D), jnp.float32),
            jax.ShapeDtypeStruct((NC, N_PAD), jnp.float32),
        ),
        mesh=mesh,
        scratch_types=(
            [pltpu.VMEM((CH,), jnp.int32) for _ in range(NB)]
            + [pltpu.VMEM((CSLAB * CH,), jnp.int32)]
            + [pltpu.VMEM((CH, D), jnp.float32) for _ in range(NB)]
            + [
                pltpu.VMEM((ONES_LEN,), jnp.float32),
                pltpu.VMEM((ONES_LEN,), jnp.float32),
                pltpu.VMEM_SHARED((N_PAD, D), jnp.float32),
                pltpu.VMEM_SHARED((N_PAD,), jnp.float32),
            ]
            + [pltpu.SemaphoreType.DMA for _ in range(3 * NB)]
        ),
    )
    def sc_kernel(x_hbm, e_hbm, s_out, deg_out, *scr):
        rowb = scr[0:NB]
        colslab = scr[NB]
        gbuf = scr[NB + 1:2 * NB + 1]
        ones_v, z80, acc, dacc = scr[2 * NB + 1:2 * NB + 5]
        semi = scr[2 * NB + 5:3 * NB + 5]
        semg = scr[3 * NB + 5:4 * NB + 5]
        sems = scr[4 * NB + 5:5 * NB + 5]

        c = lax.axis_index("c")
        s = lax.axis_index("s")
        wid = s * NC + c
        ebase = wid * EPW

        zero16 = jnp.zeros((L,), jnp.float32)
        one16 = jnp.ones((L,), jnp.float32)

        for i in range(ONES_LEN // L):
            ones_v[pl.ds(i * L, L)] = one16
            z80[pl.ds(i * L, L)] = zero16

        def fill_buf(i, carry):
            for j in range(D // L):
                gbuf[0][i, pl.ds(j * L, L)] = zero16
            return carry
        lax.fori_loop(0, CH, fill_buf, 0)

        cbase = N_EDGES + ebase
        pltpu.async_copy(e_hbm.at[pl.ds(cbase, CSLAB * CH)], colslab, semg[0])

        base = s * ROWS_PER_TILE
        for kk in range(ROWS_PER_TILE // CH):
            pltpu.async_copy(gbuf[0], acc.at[pl.ds(base + kk * CH, CH)],
                             semg[1 + (kk % 2)])
            pltpu.async_copy(z80.at[pl.ds(0, CH)], dacc.at[pl.ds(base + kk * CH, CH)],
                             semg[3])
        pltpu.make_async_copy(e_hbm.at[pl.ds(cbase, CSLAB * CH)], colslab,
                              semg[0]).wait()
        for kk in range(ROWS_PER_TILE // CH):
            pltpu.make_async_copy(gbuf[0], acc.at[pl.ds(base, CH)],
                                  semg[1 + (kk % 2)]).wait()
            pltpu.make_async_copy(z80.at[pl.ds(0, CH)], dacc.at[pl.ds(base, CH)],
                                  semg[3]).wait()

        plsc.subcore_barrier()

        def idx_start(cc, b):
            eb = ebase + cc * CH
            pltpu.async_copy(e_hbm.at[pl.ds(eb, CH)], rowb[b], semi[b])

        def idx_wait(cc, b):
            eb = ebase + cc * CH
            pltpu.make_async_copy(e_hbm.at[pl.ds(eb, CH)], rowb[b], semi[b]).wait()

        def cidx(cc, coff):
            return colslab.at[pl.ds((cc - coff) * CH, CH)]

        def gather_start(cc, b, coff):
            pltpu.async_copy(x_hbm.at[cidx(cc, coff)], gbuf[b], semg[b])

        def gather_wait(cc, b, coff):
            pltpu.make_async_copy(x_hbm.at[cidx(cc, coff)], gbuf[b], semg[b]).wait()

        def scatter_start(b):
            pltpu.async_copy(gbuf[b], acc.at[rowb[b]], sems[b], add=True)
            pltpu.async_copy(ones_v.at[pl.ds(0, CH)], dacc.at[rowb[b]], sems[b], add=True)

        def scatter_wait(b):
            pltpu.make_async_copy(gbuf[b], acc.at[rowb[b]], sems[b]).wait()
            pltpu.make_async_copy(ones_v.at[pl.ds(0, CH)], dacc.at[rowb[b]], sems[b]).wait()

        for b in range(NB):
            idx_start(b, b)

        def make_group_body(coff):
            def group_body(g, carry):
                cg = g * NB
                for b in range(NB):
                    idx_wait(cg + b, b)
                    gather_start(cg + b, b, coff)
                for b in range(NB):
                    gather_wait(cg + b, b, coff)
                    scatter_start(b)
                for b in range(NB):
                    scatter_wait(b)
                    idx_start(cg + NB + b, b)
                return carry
            return group_body

        lax.fori_loop(0, NG_A, make_group_body(0), 0)
        nb_rest = (NCHUNK - CSLAB) * CH
        pltpu.sync_copy(e_hbm.at[pl.ds(cbase + CSLAB * CH, nb_rest)],
                        colslab.at[pl.ds(0, nb_rest)])
        lax.fori_loop(NG_A, NG, make_group_body(CSLAB), 0)

        ce = NG * NB
        for b in range(NB):
            idx_wait(ce + b, b)
            gather_start(ce + b, b, CSLAB)
        for b in range(NB):
            gather_wait(ce + b, b, CSLAB)
            scatter_start(b)
        LEFT = NCHUNK - (NG + 1) * NB
        for j in range(LEFT):
            cl = (NG + 1) * NB + j
            scatter_wait(j)
            pltpu.sync_copy(e_hbm.at[pl.ds(ebase + cl * CH, CH)], rowb[j])
            pltpu.async_copy(x_hbm.at[cidx(cl, CSLAB)], gbuf[j], semg[j]).wait()
            pltpu.sync_copy(gbuf[j], acc.at[rowb[j]], add=True)
            pltpu.sync_copy(ones_v.at[pl.ds(0, CH)], dacc.at[rowb[j]], add=True)
        for b in range(LEFT, NB):
            scatter_wait(b)

        plsc.subcore_barrier()

        pltpu.sync_copy(acc.at[pl.ds(base, ROWS_PER_TILE)],
                        s_out.at[c, pl.ds(base, ROWS_PER_TILE)])
        pltpu.sync_copy(dacc.at[pl.ds(base, ROWS_PER_TILE)],
                        deg_out.at[c, pl.ds(base, ROWS_PER_TILE)])

    return sc_kernel(x, eflat)


def _tc_precompute(x, W, b2):
    BLK = 1000

    def body(x_ref, w_ref, b_ref, p_ref):
        Wa = w_ref[0:D, :]
        Wb = w_ref[D:2 * D, :]
        p_ref[...] = jnp.dot(x_ref[...], Wa - Wb,
                             preferred_element_type=jnp.float32) + b_ref[...]

    return pl.pallas_call(
        body,
        grid=(N_NODES // BLK,),
        in_specs=[
            pl.BlockSpec((BLK, D), lambda i: (i, 0)),
            pl.BlockSpec((2 * D, D), lambda i: (0, 0)),
            pl.BlockSpec((1, D), lambda i: (0, 0)),
        ],
        out_specs=pl.BlockSpec((BLK, D), lambda i: (i, 0)),
        out_shape=jax.ShapeDtypeStruct((N_NODES, D), jnp.float32),
    )(x, W, b2)


def _tc_combine(P, W, s2, deg_col):
    BLK = 1000

    def body(p_ref, w_ref, s_ref, d_ref, o_ref):
        S = s_ref[0] + s_ref[1]
        deg = d_ref[...]
        Wb = w_ref[D:2 * D, :]
        o_ref[...] = deg * p_ref[...] + jnp.dot(
            S, Wb, preferred_element_type=jnp.float32)

    return pl.pallas_call(
        body,
        grid=(N_NODES // BLK,),
        in_specs=[
            pl.BlockSpec((BLK, D), lambda i: (i, 0)),
            pl.BlockSpec((2 * D, D), lambda i: (0, 0)),
            pl.BlockSpec((NC, BLK, D), lambda i: (0, i, 0)),
            pl.BlockSpec((BLK, 1), lambda i: (i, 0)),
        ],
        out_specs=pl.BlockSpec((BLK, D), lambda i: (i, 0)),
        out_shape=jax.ShapeDtypeStruct((N_NODES, D), jnp.float32),
    )(P, W, s2, deg_col)


@jax.jit
def kernel(x, edge_index, W, b):
    eflat = edge_index.astype(jnp.int32).reshape(2 * N_EDGES)
    s2, d2 = _sc_scatter(x, eflat)
    P = _tc_precompute(x, W, b.reshape(1, D))
    deg_col = (d2[0, :N_NODES] + d2[1, :N_NODES]).reshape(N_NODES, 1)
    return _tc_combine(P, W, s2, deg_col)


# speedup vs baseline: 16.6948x; 1.0150x over previous
import functools

import jax
import jax.numpy as jnp
from jax import lax
from jax.experimental import pallas as pl
from jax.experimental.pallas import tpu as pltpu
from jax.experimental.pallas import tpu_sc as plsc

N_NODES = 10000
N_EDGES = 320000
D = 128

NC, NS, L = 2, 16, 16
NW = NC * NS
EPW = N_EDGES // NW
CH = 40
NCHUNK = EPW // CH
NB = 8
NG = NCHUNK // NB - 1
CSLAB = 128
NG_A = CSLAB // NB
ONES_LEN = ((CH + 15) // 16) * 16
N_PAD = 10240
ROWS_PER_TILE = N_PAD // NS


def _sc_scatter(x, eflat):
    mesh = plsc.VectorSubcoreMesh(core_axis_name="c", subcore_axis_name="s")

    @functools.partial(
        pl.kernel,
        out_type=(
            jax.ShapeDtypeStruct((NC, N_PAD, D), jnp.float32),
            jax.ShapeDtypeStruct((NC, N_PAD), jnp.float32),
        ),
        mesh=mesh,
        scratch_types=(
            [pltpu.VMEM((CH,), jnp.int32) for _ in range(NB)]
            + [pltpu.VMEM((CSLAB * CH,), jnp.int32)]
            + [pltpu.VMEM((CH, D), jnp.float32) for _ in range(NB)]
            + [
                pltpu.VMEM((ONES_LEN,), jnp.float32),
                pltpu.VMEM((ONES_LEN,), jnp.float32),
                pltpu.VMEM_SHARED((N_PAD, D), jnp.float32),
                pltpu.VMEM_SHARED((N_PAD,), jnp.float32),
            ]
            + [pltpu.SemaphoreType.DMA for _ in range(3 * NB)]
        ),
    )
    def sc_kernel(x_hbm, e_hbm, s_out, deg_out, *scr):
        rowb = scr[0:NB]
        colslab = scr[NB]
        gbuf = scr[NB + 1:2 * NB + 1]
        ones_v, z80, acc, dacc = scr[2 * NB + 1:2 * NB + 5]
        semi = scr[2 * NB + 5:3 * NB + 5]
        semg = scr[3 * NB + 5:4 * NB + 5]
        sems = scr[4 * NB + 5:5 * NB + 5]

        c = lax.axis_index("c")
        s = lax.axis_index("s")
        wid = s * NC + c
        ebase = wid * EPW

        zero16 = jnp.zeros((L,), jnp.float32)
        one16 = jnp.ones((L,), jnp.float32)

        cbase = N_EDGES + ebase
        pltpu.async_copy(e_hbm.at[pl.ds(cbase, CSLAB * CH)], colslab, semg[NB - 1])
        for b in range(NB):
            eb = ebase + b * CH
            pltpu.async_copy(e_hbm.at[pl.ds(eb, CH)], rowb[b], semi[b])

        for i in range(ONES_LEN // L):
            ones_v[pl.ds(i * L, L)] = one16
            z80[pl.ds(i * L, L)] = zero16

        def fill_buf(i, carry):
            for j in range(D // L):
                gbuf[0][i, pl.ds(j * L, L)] = zero16
            return carry
        lax.fori_loop(0, CH, fill_buf, 0)

        base = s * ROWS_PER_TILE
        for kk in range(ROWS_PER_TILE // CH):
            pltpu.async_copy(gbuf[0], acc.at[pl.ds(base + kk * CH, CH)],
                             semg[1 + (kk % 2)])
            pltpu.async_copy(z80.at[pl.ds(0, CH)], dacc.at[pl.ds(base + kk * CH, CH)],
                             semg[3])
        pltpu.make_async_copy(e_hbm.at[pl.ds(cbase, CSLAB * CH)], colslab,
                              semg[NB - 1]).wait()
        for kk in range(ROWS_PER_TILE // CH):
            pltpu.make_async_copy(gbuf[0], acc.at[pl.ds(base, CH)],
                                  semg[1 + (kk % 2)]).wait()
            pltpu.make_async_copy(z80.at[pl.ds(0, CH)], dacc.at[pl.ds(base, CH)],
                                  semg[3]).wait()

        plsc.subcore_barrier()

        def idx_start(cc, b):
            eb = ebase + cc * CH
            pltpu.async_copy(e_hbm.at[pl.ds(eb, CH)], rowb[b], semi[b])

        def idx_wait(cc, b):
            eb = ebase + cc * CH
            pltpu.make_async_copy(e_hbm.at[pl.ds(eb, CH)], rowb[b], semi[b]).wait()

        def cidx(cc, coff):
            return colslab.at[pl.ds((cc - coff) * CH, CH)]

        def gather_start(cc, b, coff):
            pltpu.async_copy(x_hbm.at[cidx(cc, coff)], gbuf[b], semg[b])

        def gather_wait(cc, b, coff):
            pltpu.make_async_copy(x_hbm.at[cidx(cc, coff)], gbuf[b], semg[b]).wait()

        def scatter_start(b):
            pltpu.async_copy(gbuf[b], acc.at[rowb[b]], sems[b], add=True)
            pltpu.async_copy(ones_v.at[pl.ds(0, CH)], dacc.at[rowb[b]], sems[b], add=True)

        def scatter_wait(b):
            pltpu.make_async_copy(gbuf[b], acc.at[rowb[b]], sems[b]).wait()
            pltpu.make_async_copy(ones_v.at[pl.ds(0, CH)], dacc.at[rowb[b]], sems[b]).wait()

        def make_group_body(coff):
            def group_body(g, carry):
                cg = g * NB
                for b in range(NB):
                    idx_wait(cg + b, b)
                    gather_start(cg + b, b, coff)
                for b in range(NB):
                    gather_wait(cg + b, b, coff)
                    scatter_start(b)
                for b in range(NB):
                    scatter_wait(b)
                    idx_start(cg + NB + b, b)
                return carry
            return group_body

        lax.fori_loop(0, NG_A, make_group_body(0), 0)
        nb_rest = (NCHUNK - CSLAB) * CH
        pltpu.sync_copy(e_hbm.at[pl.ds(cbase + CSLAB * CH, nb_rest)],
                        colslab.at[pl.ds(0, nb_rest)])
        lax.fori_loop(NG_A, NG, make_group_body(CSLAB), 0)

        ce = NG * NB
        for b in range(NB):
            idx_wait(ce + b, b)
            gather_start(ce + b, b, CSLAB)
        for b in range(NB):
            gather_wait(ce + b, b, CSLAB)
            scatter_start(b)
        LEFT = NCHUNK - (NG + 1) * NB
        for j in range(LEFT):
            cl = (NG + 1) * NB + j
            scatter_wait(j)
            pltpu.sync_copy(e_hbm.at[pl.ds(ebase + cl * CH, CH)], rowb[j])
            pltpu.async_copy(x_hbm.at[cidx(cl, CSLAB)], gbuf[j], semg[j]).wait()
            pltpu.sync_copy(gbuf[j], acc.at[rowb[j]], add=True)
            pltpu.sync_copy(ones_v.at[pl.ds(0, CH)], dacc.at[rowb[j]], add=True)
        for b in range(LEFT, NB):
            scatter_wait(b)

        plsc.subcore_barrier()

        pltpu.sync_copy(acc.at[pl.ds(base, ROWS_PER_TILE)],
                        s_out.at[c, pl.ds(base, ROWS_PER_TILE)])
        pltpu.sync_copy(dacc.at[pl.ds(base, ROWS_PER_TILE)],
                        deg_out.at[c, pl.ds(base, ROWS_PER_TILE)])

    return sc_kernel(x, eflat)


def _tc_precompute(x, W, b2):
    BLK = 1000

    def body(x_ref, w_ref, b_ref, p_ref):
        Wa = w_ref[0:D, :]
        Wb = w_ref[D:2 * D, :]
        p_ref[...] = jnp.dot(x_ref[...], Wa - Wb,
                             preferred_element_type=jnp.float32) + b_ref[...]

    return pl.pallas_call(
        body,
        grid=(N_NODES // BLK,),
        in_specs=[
            pl.BlockSpec((BLK, D), lambda i: (i, 0)),
            pl.BlockSpec((2 * D, D), lambda i: (0, 0)),
            pl.BlockSpec((1, D), lambda i: (0, 0)),
        ],
        out_specs=pl.BlockSpec((BLK, D), lambda i: (i, 0)),
        out_shape=jax.ShapeDtypeStruct((N_NODES, D), jnp.float32),
    )(x, W, b2)


def _tc_combine(P, W, s2, deg_col):
    BLK = 1000

    def body(p_ref, w_ref, s_ref, d_ref, o_ref):
        S = s_ref[0] + s_ref[1]
        deg = d_ref[...]
        Wb = w_ref[D:2 * D, :]
        o_ref[...] = deg * p_ref[...] + jnp.dot(
            S, Wb, preferred_element_type=jnp.float32)

    return pl.pallas_call(
        body,
        grid=(N_NODES // BLK,),
        in_specs=[
            pl.BlockSpec((BLK, D), lambda i: (i, 0)),
            pl.BlockSpec((2 * D, D), lambda i: (0, 0)),
            pl.BlockSpec((NC, BLK, D), lambda i: (0, i, 0)),
            pl.BlockSpec((BLK, 1), lambda i: (i, 0)),
        ],
        out_specs=pl.BlockSpec((BLK, D), lambda i: (i, 0)),
        out_shape=jax.ShapeDtypeStruct((N_NODES, D), jnp.float32),
    )(P, W, s2, deg_col)


@jax.jit
def kernel(x, edge_index, W, b):
    eflat = edge_index.astype(jnp.int32).reshape(2 * N_EDGES)
    s2, d2 = _sc_scatter(x, eflat)
    P = _tc_precompute(x, W, b.reshape(1, D))
    deg_col = (d2[0, :N_NODES] + d2[1, :N_NODES]).reshape(N_NODES, 1)
    return _tc_combine(P, W, s2, deg_col)
